# Initial kernel scaffold; baseline (speedup 1.0000x reference)
import jax
import jax.numpy as jnp
from jax.experimental import pallas as pl


def kernel(loc_data, conf_data, prior_data):
    raise NotImplementedError("write your pallas kernel here")


# speedup vs baseline: 7.6890x; 7.6890x over previous
import struct

import jax
import jax.numpy as jnp
from jax import lax
from jax.experimental import pallas as pl
from jax.experimental.pallas import tpu as pltpu
from jax.experimental.pallas import tpu_sc as plsc

L = 16
B = 8
P = 20000
NCLS = 21
NPROB = B * (NCLS - 1)
PV = P // L
CAP = 512
NU = CAP // L
CH = 2000
NCHUNK = P // CH
CV = CH // L
NMS_TOP_K = 400
TOP_K = 200
CONF_THRESH = 0.01
NMS_THRESH = 0.45
V0 = 0.1
V1 = 0.2


def _f32_bits(x):
    return struct.unpack("<i", struct.pack("<f", x))[0]


_B001 = _f32_bits(CONF_THRESH)
_B97 = _f32_bits(0.97)
_B98 = _f32_bits(0.98)
_B99 = _f32_bits(0.99)
_BINF = 0x7F800000

_LAYERS = [(k, j)
           for k in (2, 4, 8, 16, 32)
           for j in (k // 2, k // 4, k // 8, k // 16, k // 32) if j >= 1]


def _iota():
    return lax.iota(jnp.int32, L)


def _full_i(x):
    return jnp.full((L,), x, jnp.int32)


def _lane0(v):
    return lax.squeeze(lax.slice(v, (0,), (1,)), dimensions=(0,))


def _popcnt(m):
    return _lane0(plsc.all_reduce_population_count(m))


def _detect_body(conf_hbm, dx_hbm, dy_hbm, dw_hbm, dh_hbm,
                 px_hbm, py_hbm, pw_hbm, ph_hbm, out_hbm,
                 scores, x1f, y1f, x2f, y2f,
                 ldx, ldy, ldw, ldh, lpx, lpy, lpw, lph,
                 cand_s, cand_i, skey, srt_s, srt_i,
                 x1c, y1c, x2c, y2c, ac, supp, outbuf):
    info = plsc.get_sparse_core_info()
    nc = info.num_cores
    wid = lax.axis_index("s") * nc + lax.axis_index("c")
    nper = NPROB // (nc * info.num_subcores)
    b = (wid * nper) // (NCLS - 1)
    iota = _iota()

    def count_gt(bits):
        thr = lax.bitcast_convert_type(_full_i(bits), jnp.float32)

        def cbody(k, acc):
            s = scores[pl.ds(k * L, L)]
            return acc + plsc.all_reduce_population_count(s > thr)

        acc = lax.fori_loop(0, PV, cbody, jnp.zeros((L,), jnp.int32))
        return _lane0(acc)

    def zero_outbuf():
        def zb(k, _):
            outbuf[pl.ds(k * L, L)] = jnp.zeros((L,), jnp.float32)
            return 0
        lax.fori_loop(0, outbuf.shape[0] // L, zb, 0)

    @pl.when(wid < B)
    def _():
        zero_outbuf()
        pltpu.sync_copy(outbuf.at[pl.ds(0, TOP_K * 5)],
                        out_hbm.at[pl.ds(wid * NCLS * TOP_K * 5, TOP_K * 5)])

    def chunk(ci, _):
        off = b * P + ci * CH
        poff = ci * CH
        pltpu.sync_copy(dx_hbm.at[pl.ds(off, CH)], ldx)
        pltpu.sync_copy(dy_hbm.at[pl.ds(off, CH)], ldy)
        pltpu.sync_copy(dw_hbm.at[pl.ds(off, CH)], ldw)
        pltpu.sync_copy(dh_hbm.at[pl.ds(off, CH)], ldh)
        pltpu.sync_copy(px_hbm.at[pl.ds(poff, CH)], lpx)
        pltpu.sync_copy(py_hbm.at[pl.ds(poff, CH)], lpy)
        pltpu.sync_copy(pw_hbm.at[pl.ds(poff, CH)], lpw)
        pltpu.sync_copy(ph_hbm.at[pl.ds(poff, CH)], lph)

        def dec(k, _):
            sl = pl.ds(k * L, L)
            dx = ldx[sl]
            dy = ldy[sl]
            dw = ldw[sl]
            dh = ldh[sl]
            px = lpx[sl]
            py = lpy[sl]
            pw = lpw[sl]
            ph = lph[sl]
            cx = px + dx * jnp.float32(V0) * pw
            cy = py + dy * jnp.float32(V0) * ph
            w = pw * jnp.exp(dw * jnp.float32(V1))
            h = ph * jnp.exp(dh * jnp.float32(V1))
            x1 = cx - w / 2.0
            y1 = cy - h / 2.0
            osl = pl.ds(poff + k * L, L)
            x1f[osl] = x1
            y1f[osl] = y1
            x2f[osl] = x1 + w
            y2f[osl] = y1 + h
            return 0

        lax.fori_loop(0, CV, dec, 0)
        return 0

    lax.fori_loop(0, NCHUNK, chunk, 0)

    def problem(q, _):
        prob = wid * nper + q
        c = prob % (NCLS - 1)

        pltpu.sync_copy(conf_hbm.at[prob], scores)

        t001 = lax.bitcast_convert_type(_full_i(_B001), jnp.float32)
        t97 = lax.bitcast_convert_type(_full_i(_B97), jnp.float32)
        t98 = lax.bitcast_convert_type(_full_i(_B98), jnp.float32)
        t99 = lax.bitcast_convert_type(_full_i(_B99), jnp.float32)

        def p1(k, acc):
            a0, a1, a2, a3 = acc
            s = scores[pl.ds(k * L, L)]
            a0 = a0 + plsc.all_reduce_population_count(s > t001)
            a1 = a1 + plsc.all_reduce_population_count(s > t97)
            a2 = a2 + plsc.all_reduce_population_count(s > t98)
            a3 = a3 + plsc.all_reduce_population_count(s > t99)
            return a0, a1, a2, a3

        z = jnp.zeros((L,), jnp.int32)
        a0, a1, a2, a3 = lax.fori_loop(0, PV, p1, (z, z, z, z))
        c001 = _lane0(a0)
        c97 = _lane0(a1)
        c98 = _lane0(a2)
        c99 = _lane0(a3)

        K = jnp.int32(NMS_TOP_K)
        lo_b = jnp.int32(_B001)
        cnt_lo = c001
        lo_b = jnp.where(c97 >= K, jnp.int32(_B97), lo_b)
        cnt_lo = jnp.where(c97 >= K, c97, cnt_lo)
        lo_b = jnp.where(c98 >= K, jnp.int32(_B98), lo_b)
        cnt_lo = jnp.where(c98 >= K, c98, cnt_lo)
        lo_b = jnp.where(c99 >= K, jnp.int32(_B99), lo_b)
        cnt_lo = jnp.where(c99 >= K, c99, cnt_lo)
        hi_b = jnp.where(c99 < K, jnp.int32(_B99), jnp.int32(_BINF))
        hi_b = jnp.where(c98 < K, jnp.int32(_B98), hi_b)
        hi_b = jnp.where(c97 < K, jnp.int32(_B97), hi_b)

        def s_cond(st):
            lo, hi, cl = st
            return (cl > jnp.int32(CAP)) & (hi - lo > 1)

        def s_body(st):
            lo, hi, cl = st
            mid = lo + (hi - lo) // 2
            cm = count_gt(mid)
            lo2 = jnp.where(cm >= K, mid, lo)
            cl2 = jnp.where(cm >= K, cm, cl)
            hi2 = jnp.where(cm >= K, hi, mid)
            return lo2, hi2, cl2

        lo_b, hi_b, cnt_lo = lax.while_loop(s_cond, s_body,
                                            (lo_b, hi_b, cnt_lo))
        thr = lax.bitcast_convert_type(_full_i(lo_b), jnp.float32)

        def coll(k, st):
            cnt, mnb, mxb = st
            s = scores[pl.ds(k * L, L)]
            m = s > thr

            @pl.when(cnt < jnp.int32(CAP))
            def _():
                plsc.store_compressed(cand_s.at[pl.ds(cnt, L)], s, mask=m)
                plsc.store_compressed(cand_i.at[pl.ds(cnt, L)],
                                      iota + k * L, mask=m)

            sb = lax.bitcast_convert_type(s, jnp.int32)
            mnb = jnp.minimum(mnb, jnp.where(m, sb, jnp.int32(0x7F7FFFFF)))
            mxb = jnp.maximum(mxb, jnp.where(m, sb, jnp.int32(0)))
            cnt = cnt + _popcnt(m)
            return cnt, mnb, mxb

        cnt, mnb_v, mxb_v = lax.fori_loop(
            0, PV, coll,
            (jnp.int32(0), _full_i(0x7F7FFFFF), _full_i(0)))
        M = jnp.minimum(cnt, jnp.int32(CAP))
        mn_s, _d0 = plsc.sort_key_val(mnb_v, mnb_v)
        mx_s, _d1 = plsc.sort_key_val(mxb_v, mxb_v, descending=True)
        mnb = _lane0(mn_s)
        mxb = _lane0(mx_s)
        rng = mxb - mnb

        def sh_cond(sh):
            return lax.shift_right_arithmetic(rng, sh) >= jnp.int32(1 << 22)

        shift = lax.while_loop(sh_cond, lambda sh: sh + 1, jnp.int32(0))
        shv = _full_i(0) + shift

        def mkkey(k, _):
            base = k * L
            ids = iota + base
            valid = ids < M
            sb = lax.bitcast_convert_type(cand_s[pl.ds(base, L)], jnp.int32)
            d = lax.shift_right_arithmetic(sb - mnb, shv)
            key = jnp.bitwise_or(lax.shift_left(d, _full_i(9)),
                                 jnp.int32(CAP - 1) - ids)
            key = jnp.where(valid, key + 1, 0)
            sk, _sv = plsc.sort_key_val(key, key, descending=True)
            skey[pl.ds(base, L)] = sk
            return 0

        lax.fori_loop(0, NU, mkkey, 0)

        for (kk, jj) in _LAYERS:
            def net(i, _, kk=kk, jj=jj):
                l = jnp.bitwise_xor(i, jnp.int32(jj))

                @pl.when(l > i)
                def _():
                    a = skey[pl.ds(i * L, L)]
                    bb = skey[pl.ds(l * L, L)]
                    rb = lax.rev(bb, (0,))
                    hi = jnp.maximum(a, rb)
                    lo = jnp.minimum(a, rb)
                    hi_s, _h = plsc.sort_key_val(hi, hi, descending=True)
                    lo_s, _l = plsc.sort_key_val(lo, lo, descending=True)
                    maxfirst = jnp.bitwise_and(i, jnp.int32(kk)) == 0
                    skey[pl.ds(i * L, L)] = jnp.where(maxfirst, hi_s, lo_s)
                    skey[pl.ds(l * L, L)] = jnp.where(maxfirst, lo_s, hi_s)

                return 0

            lax.fori_loop(0, NU, net, 0)

        def unp(k, _):
            base = k * L
            sk = skey[pl.ds(base, L)]
            ordv = jnp.where(sk > 0,
                             jnp.int32(CAP - 1) -
                             jnp.bitwise_and(sk - 1, jnp.int32(CAP - 1)),
                             0)
            srt_s[pl.ds(base, L)] = plsc.load_gather(cand_s, [ordv])
            pi = plsc.load_gather(cand_i, [ordv])
            pi = jnp.minimum(jnp.maximum(pi, 0), jnp.int32(P - 1))
            srt_i[pl.ds(base, L)] = pi
            x1 = plsc.load_gather(x1f, [pi])
            y1 = plsc.load_gather(y1f, [pi])
            x2 = plsc.load_gather(x2f, [pi])
            y2 = plsc.load_gather(y2f, [pi])
            x1c[pl.ds(base, L)] = x1
            y1c[pl.ds(base, L)] = y1
            x2c[pl.ds(base, L)] = x2
            y2c[pl.ds(base, L)] = y2
            ac[pl.ds(base, L)] = (x2 - x1) * (y2 - y1)
            supp[pl.ds(base, L)] = jnp.zeros((L,), jnp.float32)
            return 0

        lax.fori_loop(0, NU, unp, 0)

        zero_outbuf()
        Mc = jnp.minimum(M, jnp.int32(NMS_TOP_K))
        NB = (Mc + (L - 1)) // L

        def splat(ref, i):
            return plsc.load_gather(ref, [_full_i(0) + i])

        def n_cond(st):
            i, nk = st
            return (i < Mc) & (nk < jnp.int32(TOP_K))

        def n_body(st):
            i, nk = st
            sup_i = _lane0(splat(supp, i))

            def keep(opn):
                i, nk = opn
                sv = splat(srt_s, i)
                x1i = splat(x1c, i)
                y1i = splat(y1c, i)
                x2i = splat(x2c, i)
                y2i = splat(y2c, i)
                ai = splat(ac, i)
                oidx = nk * 5 + iota
                val = sv
                val = jnp.where(iota == 1, x1i, val)
                val = jnp.where(iota == 2, y1i, val)
                val = jnp.where(iota == 3, x2i, val)
                val = jnp.where(iota == 4, y2i, val)
                plsc.store_scatter(outbuf, [oidx], val, mask=iota < 5)

                iv = _full_i(0) + i
                kb = i // L

                def sup_blk(k, _):
                    base = k * L
                    jids = iota + base
                    bx1 = x1c[pl.ds(base, L)]
                    by1 = y1c[pl.ds(base, L)]
                    bx2 = x2c[pl.ds(base, L)]
                    by2 = y2c[pl.ds(base, L)]
                    ba = ac[pl.ds(base, L)]
                    ix1 = jnp.maximum(x1i, bx1)
                    iy1 = jnp.maximum(y1i, by1)
                    ix2 = jnp.minimum(x2i, bx2)
                    iy2 = jnp.minimum(y2i, by2)
                    inter = (jnp.maximum(ix2 - ix1, 0.0) *
                             jnp.maximum(iy2 - iy1, 0.0))
                    union = ai + ba - inter
                    iou = inter / jnp.maximum(union, jnp.float32(1e-12))
                    sup = (iou > jnp.float32(NMS_THRESH)) & (jids > iv)
                    sp = supp[pl.ds(base, L)]
                    supp[pl.ds(base, L)] = jnp.where(sup, 1.0, sp)
                    return 0

                lax.fori_loop(kb, NB, sup_blk, 0)
                return i + 1, nk + 1

            def skip(opn):
                i, nk = opn
                return i + 1, nk

            return lax.cond(sup_i == 0.0, keep, skip, (i, nk))

        lax.while_loop(n_cond, n_body, (jnp.int32(0), jnp.int32(0)))

        off = (b * NCLS + c + 1) * (TOP_K * 5)
        pltpu.sync_copy(outbuf.at[pl.ds(0, TOP_K * 5)],
                        out_hbm.at[pl.ds(off, TOP_K * 5)])
        return 0

    lax.fori_loop(0, nper, problem, 0)


@jax.jit
def _detect(conf_t, dx, dy, dw, dh, px, py, pw, ph):
    mesh = plsc.VectorSubcoreMesh(core_axis_name="c", subcore_axis_name="s")
    f = pl.kernel(
        _detect_body,
        out_type=jax.ShapeDtypeStruct((B * NCLS * TOP_K * 5,), jnp.float32),
        mesh=mesh,
        compiler_params=pltpu.CompilerParams(needs_layout_passes=False,
                                             use_tc_tiling_on_sc=False),
        scratch_types=[
            pltpu.VMEM((P,), jnp.float32),
            pltpu.VMEM((P,), jnp.float32),
            pltpu.VMEM((P,), jnp.float32),
            pltpu.VMEM((P,), jnp.float32),
            pltpu.VMEM((P,), jnp.float32),
            pltpu.VMEM((CH,), jnp.float32),
            pltpu.VMEM((CH,), jnp.float32),
            pltpu.VMEM((CH,), jnp.float32),
            pltpu.VMEM((CH,), jnp.float32),
            pltpu.VMEM((CH,), jnp.float32),
            pltpu.VMEM((CH,), jnp.float32),
            pltpu.VMEM((CH,), jnp.float32),
            pltpu.VMEM((CH,), jnp.float32),
            pltpu.VMEM((CAP + L,), jnp.float32),
            pltpu.VMEM((CAP + L,), jnp.int32),
            pltpu.VMEM((CAP,), jnp.int32),
            pltpu.VMEM((CAP,), jnp.float32),
            pltpu.VMEM((CAP,), jnp.int32),
            pltpu.VMEM((CAP,), jnp.float32),
            pltpu.VMEM((CAP,), jnp.float32),
            pltpu.VMEM((CAP,), jnp.float32),
            pltpu.VMEM((CAP,), jnp.float32),
            pltpu.VMEM((CAP,), jnp.float32),
            pltpu.VMEM((CAP,), jnp.float32),
            pltpu.VMEM((TOP_K * 5 + L - (TOP_K * 5) % L,), jnp.float32),
        ],
    )
    return f(conf_t, dx, dy, dw, dh, px, py, pw, ph)


def kernel(loc_data, conf_data, prior_data):
    conf_t = (conf_data.reshape(B, P, NCLS)
              .transpose(0, 2, 1)[:, 1:, :]
              .reshape(NPROB, P))
    loc4 = loc_data.reshape(B * P, 4)
    dx = loc4[:, 0]
    dy = loc4[:, 1]
    dw = loc4[:, 2]
    dh = loc4[:, 3]
    px = prior_data[:, 0]
    py = prior_data[:, 1]
    pw = prior_data[:, 2]
    ph = prior_data[:, 3]
    out = _detect(conf_t, dx, dy, dw, dh, px, py, pw, ph)
    return out.reshape(B, NCLS, TOP_K, 5)


# speedup vs baseline: 8.0895x; 1.0521x over previous
import struct

import jax
import jax.numpy as jnp
from jax import lax
from jax.experimental import pallas as pl
from jax.experimental.pallas import tpu as pltpu
from jax.experimental.pallas import tpu_sc as plsc

L = 16
B = 8
P = 20000
NCLS = 21
NPROB = B * (NCLS - 1)
PV = P // L
CAP = 512
NU = CAP // L
CH = 2000
NCHUNK = P // CH
CV = CH // L
NMS_TOP_K = 400
TOP_K = 200
CONF_THRESH = 0.01
NMS_THRESH = 0.45
V0 = 0.1
V1 = 0.2


def _f32_bits(x):
    return struct.unpack("<i", struct.pack("<f", x))[0]


_B001 = _f32_bits(CONF_THRESH)
_B97 = _f32_bits(0.97)
_B98 = _f32_bits(0.98)
_B99 = _f32_bits(0.99)
_BINF = 0x7F800000

_LAYERS = [(k, j)
           for k in (2, 4, 8, 16, 32)
           for j in (k // 2, k // 4, k // 8, k // 16, k // 32) if j >= 1]


def _iota():
    return lax.iota(jnp.int32, L)


def _full_i(x):
    return jnp.full((L,), x, jnp.int32)


def _lane0(v):
    return lax.squeeze(lax.slice(v, (0,), (1,)), dimensions=(0,))


def _popcnt(m):
    return _lane0(plsc.all_reduce_population_count(m))


def _detect_body(conf_hbm, dx_hbm, dy_hbm, dw_hbm, dh_hbm,
                 px_hbm, py_hbm, pw_hbm, ph_hbm, out_hbm,
                 scores, x1f, y1f, x2f, y2f,
                 ldx, ldy, ldw, ldh, lpx, lpy, lpw, lph,
                 cand_s, cand_i, skey, srt_s, srt_i,
                 x1c, y1c, x2c, y2c, ac, supp, outbuf):
    info = plsc.get_sparse_core_info()
    nc = info.num_cores
    wid = lax.axis_index("s") * nc + lax.axis_index("c")
    nper = NPROB // (nc * info.num_subcores)
    b = (wid * nper) // (NCLS - 1)
    iota = _iota()

    def count_gt(bits):
        thr = lax.bitcast_convert_type(_full_i(bits), jnp.float32)

        def cbody(k, acc):
            a0, a1 = acc
            a0 = a0 + plsc.all_reduce_population_count(
                scores[pl.ds(2 * k * L, L)] > thr)
            a1 = a1 + plsc.all_reduce_population_count(
                scores[pl.ds((2 * k + 1) * L, L)] > thr)
            return a0, a1

        z = jnp.zeros((L,), jnp.int32)
        a0, a1 = lax.fori_loop(0, PV // 2, cbody, (z, z))
        return _lane0(a0) + _lane0(a1)

    def zero_outbuf():
        def zb(k, _):
            outbuf[pl.ds(k * L, L)] = jnp.zeros((L,), jnp.float32)
            return 0
        lax.fori_loop(0, outbuf.shape[0] // L, zb, 0)

    @pl.when(wid < B)
    def _():
        zero_outbuf()
        pltpu.sync_copy(outbuf.at[pl.ds(0, TOP_K * 5)],
                        out_hbm.at[pl.ds(wid * NCLS * TOP_K * 5, TOP_K * 5)])

    def chunk(ci, _):
        off = b * P + ci * CH
        poff = ci * CH
        pltpu.sync_copy(dx_hbm.at[pl.ds(off, CH)], ldx)
        pltpu.sync_copy(dy_hbm.at[pl.ds(off, CH)], ldy)
        pltpu.sync_copy(dw_hbm.at[pl.ds(off, CH)], ldw)
        pltpu.sync_copy(dh_hbm.at[pl.ds(off, CH)], ldh)
        pltpu.sync_copy(px_hbm.at[pl.ds(poff, CH)], lpx)
        pltpu.sync_copy(py_hbm.at[pl.ds(poff, CH)], lpy)
        pltpu.sync_copy(pw_hbm.at[pl.ds(poff, CH)], lpw)
        pltpu.sync_copy(ph_hbm.at[pl.ds(poff, CH)], lph)

        def dec(k, _):
            sl = pl.ds(k * L, L)
            dx = ldx[sl]
            dy = ldy[sl]
            dw = ldw[sl]
            dh = ldh[sl]
            px = lpx[sl]
            py = lpy[sl]
            pw = lpw[sl]
            ph = lph[sl]
            cx = px + dx * jnp.float32(V0) * pw
            cy = py + dy * jnp.float32(V0) * ph
            w = pw * jnp.exp(dw * jnp.float32(V1))
            h = ph * jnp.exp(dh * jnp.float32(V1))
            x1 = cx - w / 2.0
            y1 = cy - h / 2.0
            osl = pl.ds(poff + k * L, L)
            x1f[osl] = x1
            y1f[osl] = y1
            x2f[osl] = x1 + w
            y2f[osl] = y1 + h
            return 0

        lax.fori_loop(0, CV, dec, 0)
        return 0

    lax.fori_loop(0, NCHUNK, chunk, 0)

    def problem(q, _):
        prob = wid * nper + q
        c = prob % (NCLS - 1)

        pltpu.sync_copy(conf_hbm.at[prob], scores)

        t97 = lax.bitcast_convert_type(_full_i(_B97), jnp.float32)
        t98 = lax.bitcast_convert_type(_full_i(_B98), jnp.float32)

        def p1(k, acc):
            a0, a1, a2, a3 = acc
            s0 = scores[pl.ds(2 * k * L, L)]
            s1 = scores[pl.ds((2 * k + 1) * L, L)]
            a0 = a0 + plsc.all_reduce_population_count(s0 > t97)
            a1 = a1 + plsc.all_reduce_population_count(s0 > t98)
            a2 = a2 + plsc.all_reduce_population_count(s1 > t97)
            a3 = a3 + plsc.all_reduce_population_count(s1 > t98)
            return a0, a1, a2, a3

        z = jnp.zeros((L,), jnp.int32)
        a0, a1, a2, a3 = lax.fori_loop(0, PV // 2, p1, (z, z, z, z))
        c97 = _lane0(a0) + _lane0(a2)
        c98 = _lane0(a1) + _lane0(a3)

        K = jnp.int32(NMS_TOP_K)
        c001 = lax.cond(c97 < K, lambda: count_gt(jnp.int32(_B001)),
                        lambda: jnp.int32(0x7FFFFFFF))
        lo_b = jnp.where(c98 >= K, jnp.int32(_B98),
                         jnp.where(c97 >= K, jnp.int32(_B97),
                                   jnp.int32(_B001)))
        cnt_lo = jnp.where(c98 >= K, c98, jnp.where(c97 >= K, c97, c001))
        hi_b = jnp.where(c98 >= K, jnp.int32(_B99),
                         jnp.where(c97 >= K, jnp.int32(_B98),
                                   jnp.int32(_B97)))

        def s_cond(st):
            lo, hi, cl = st
            return (cl > jnp.int32(CAP)) & (hi - lo > 1)

        def s_body(st):
            lo, hi, cl = st
            mid = lo + (hi - lo) // 2
            cm = count_gt(mid)
            lo2 = jnp.where(cm >= K, mid, lo)
            cl2 = jnp.where(cm >= K, cm, cl)
            hi2 = jnp.where(cm >= K, hi, mid)
            return lo2, hi2, cl2

        lo_b, hi_b, cnt_lo = lax.while_loop(s_cond, s_body,
                                            (lo_b, hi_b, cnt_lo))
        thr = lax.bitcast_convert_type(_full_i(lo_b), jnp.float32)

        def coll1(k, st):
            cnt, mnb, mxb = st
            s = scores[pl.ds(k * L, L)]
            m = s > thr

            @pl.when(cnt < jnp.int32(CAP))
            def _():
                plsc.store_compressed(cand_s.at[pl.ds(cnt, L)], s, mask=m)
                plsc.store_compressed(cand_i.at[pl.ds(cnt, L)],
                                      iota + k * L, mask=m)

            sb = lax.bitcast_convert_type(s, jnp.int32)
            mnb = jnp.minimum(mnb, jnp.where(m, sb, jnp.int32(0x7F7FFFFF)))
            mxb = jnp.maximum(mxb, jnp.where(m, sb, jnp.int32(0)))
            cnt = cnt + _popcnt(m)
            return cnt, mnb, mxb

        def coll(k, st):
            return coll1(2 * k + 1, coll1(2 * k, st))

        cnt, mnb_v, mxb_v = lax.fori_loop(
            0, PV // 2, coll,
            (jnp.int32(0), _full_i(0x7F7FFFFF), _full_i(0)))
        M = jnp.minimum(cnt, jnp.int32(CAP))
        mn_s, _d0 = plsc.sort_key_val(mnb_v, mnb_v)
        mx_s, _d1 = plsc.sort_key_val(mxb_v, mxb_v, descending=True)
        mnb = _lane0(mn_s)
        mxb = _lane0(mx_s)
        rng = mxb - mnb

        def sh_cond(sh):
            return lax.shift_right_arithmetic(rng, sh) >= jnp.int32(1 << 22)

        shift = lax.while_loop(sh_cond, lambda sh: sh + 1, jnp.int32(0))
        shv = _full_i(0) + shift

        def mkkey(k, _):
            base = k * L
            ids = iota + base
            valid = ids < M
            sb = lax.bitcast_convert_type(cand_s[pl.ds(base, L)], jnp.int32)
            d = lax.shift_right_arithmetic(sb - mnb, shv)
            key = jnp.bitwise_or(lax.shift_left(d, _full_i(9)),
                                 jnp.int32(CAP - 1) - ids)
            key = jnp.where(valid, key + 1, 0)
            sk, _sv = plsc.sort_key_val(key, key, descending=True)
            skey[pl.ds(base, L)] = sk
            return 0

        lax.fori_loop(0, NU, mkkey, 0)

        for (kk, jj) in _LAYERS:
            def net(t, _, kk=kk, jj=jj):
                sh = jj.bit_length() - 1
                i = ((t >> sh) << (sh + 1)) | (t & (jj - 1))
                l = i + jj
                a = skey[pl.ds(i * L, L)]
                bb = skey[pl.ds(l * L, L)]
                rb = lax.rev(bb, (0,))
                hi = jnp.maximum(a, rb)
                lo = jnp.minimum(a, rb)
                hi_s, _h = plsc.sort_key_val(hi, hi, descending=True)
                lo_s, _l = plsc.sort_key_val(lo, lo, descending=True)
                maxfirst = jnp.bitwise_and(i, jnp.int32(kk)) == 0
                skey[pl.ds(i * L, L)] = jnp.where(maxfirst, hi_s, lo_s)
                skey[pl.ds(l * L, L)] = jnp.where(maxfirst, lo_s, hi_s)
                return 0

            lax.fori_loop(0, NU // 2, net, 0)

        def unp(k, _):
            base = k * L
            sk = skey[pl.ds(base, L)]
            ordv = jnp.where(sk > 0,
                             jnp.int32(CAP - 1) -
                             jnp.bitwise_and(sk - 1, jnp.int32(CAP - 1)),
                             0)
            srt_s[pl.ds(base, L)] = plsc.load_gather(cand_s, [ordv])
            pi = plsc.load_gather(cand_i, [ordv])
            pi = jnp.minimum(jnp.maximum(pi, 0), jnp.int32(P - 1))
            srt_i[pl.ds(base, L)] = pi
            x1 = plsc.load_gather(x1f, [pi])
            y1 = plsc.load_gather(y1f, [pi])
            x2 = plsc.load_gather(x2f, [pi])
            y2 = plsc.load_gather(y2f, [pi])
            x1c[pl.ds(base, L)] = x1
            y1c[pl.ds(base, L)] = y1
            x2c[pl.ds(base, L)] = x2
            y2c[pl.ds(base, L)] = y2
            ac[pl.ds(base, L)] = (x2 - x1) * (y2 - y1)
            supp[pl.ds(base, L)] = jnp.zeros((L,), jnp.float32)
            return 0

        lax.fori_loop(0, NU, unp, 0)

        zero_outbuf()
        Mc = jnp.minimum(M, jnp.int32(NMS_TOP_K))
        NB = (Mc + (L - 1)) // L

        def splat(ref, i):
            return plsc.load_gather(ref, [_full_i(0) + i])

        def n_cond(st):
            i, nk = st
            return (i < Mc) & (nk < jnp.int32(TOP_K))

        def n_body(st):
            i, nk = st
            sup_i = _lane0(splat(supp, i))

            def keep(opn):
                i, nk = opn
                sv = splat(srt_s, i)
                x1i = splat(x1c, i)
                y1i = splat(y1c, i)
                x2i = splat(x2c, i)
                y2i = splat(y2c, i)
                ai = splat(ac, i)
                oidx = nk * 5 + iota
                val = sv
                val = jnp.where(iota == 1, x1i, val)
                val = jnp.where(iota == 2, y1i, val)
                val = jnp.where(iota == 3, x2i, val)
                val = jnp.where(iota == 4, y2i, val)
                plsc.store_scatter(outbuf, [oidx], val, mask=iota < 5)

                iv = _full_i(0) + i
                kb = i // L

                def sup_blk(k, _):
                    base = k * L
                    jids = iota + base
                    bx1 = x1c[pl.ds(base, L)]
                    by1 = y1c[pl.ds(base, L)]
                    bx2 = x2c[pl.ds(base, L)]
                    by2 = y2c[pl.ds(base, L)]
                    ba = ac[pl.ds(base, L)]
                    ix1 = jnp.maximum(x1i, bx1)
                    iy1 = jnp.maximum(y1i, by1)
                    ix2 = jnp.minimum(x2i, bx2)
                    iy2 = jnp.minimum(y2i, by2)
                    inter = (jnp.maximum(ix2 - ix1, 0.0) *
                             jnp.maximum(iy2 - iy1, 0.0))
                    union = ai + ba - inter
                    iou = inter / jnp.maximum(union, jnp.float32(1e-12))
                    sup = (iou > jnp.float32(NMS_THRESH)) & (jids > iv)
                    sp = supp[pl.ds(base, L)]
                    supp[pl.ds(base, L)] = jnp.where(sup, 1.0, sp)
                    return 0

                lax.fori_loop(kb, NB, sup_blk, 0)
                return i + 1, nk + 1

            def skip(opn):
                i, nk = opn
                return i + 1, nk

            return lax.cond(sup_i == 0.0, keep, skip, (i, nk))

        lax.while_loop(n_cond, n_body, (jnp.int32(0), jnp.int32(0)))

        off = (b * NCLS + c + 1) * (TOP_K * 5)
        pltpu.sync_copy(outbuf.at[pl.ds(0, TOP_K * 5)],
                        out_hbm.at[pl.ds(off, TOP_K * 5)])
        return 0

    lax.fori_loop(0, nper, problem, 0)


@jax.jit
def _detect(conf_t, dx, dy, dw, dh, px, py, pw, ph):
    mesh = plsc.VectorSubcoreMesh(core_axis_name="c", subcore_axis_name="s")
    f = pl.kernel(
        _detect_body,
        out_type=jax.ShapeDtypeStruct((B * NCLS * TOP_K * 5,), jnp.float32),
        mesh=mesh,
        compiler_params=pltpu.CompilerParams(needs_layout_passes=False,
                                             use_tc_tiling_on_sc=False),
        scratch_types=[
            pltpu.VMEM((P,), jnp.float32),
            pltpu.VMEM((P,), jnp.float32),
            pltpu.VMEM((P,), jnp.float32),
            pltpu.VMEM((P,), jnp.float32),
            pltpu.VMEM((P,), jnp.float32),
            pltpu.VMEM((CH,), jnp.float32),
            pltpu.VMEM((CH,), jnp.float32),
            pltpu.VMEM((CH,), jnp.float32),
            pltpu.VMEM((CH,), jnp.float32),
            pltpu.VMEM((CH,), jnp.float32),
            pltpu.VMEM((CH,), jnp.float32),
            pltpu.VMEM((CH,), jnp.float32),
            pltpu.VMEM((CH,), jnp.float32),
            pltpu.VMEM((CAP + L,), jnp.float32),
            pltpu.VMEM((CAP + L,), jnp.int32),
            pltpu.VMEM((CAP,), jnp.int32),
            pltpu.VMEM((CAP,), jnp.float32),
            pltpu.VMEM((CAP,), jnp.int32),
            pltpu.VMEM((CAP,), jnp.float32),
            pltpu.VMEM((CAP,), jnp.float32),
            pltpu.VMEM((CAP,), jnp.float32),
            pltpu.VMEM((CAP,), jnp.float32),
            pltpu.VMEM((CAP,), jnp.float32),
            pltpu.VMEM((CAP,), jnp.float32),
            pltpu.VMEM((TOP_K * 5 + L - (TOP_K * 5) % L,), jnp.float32),
        ],
    )
    return f(conf_t, dx, dy, dw, dh, px, py, pw, ph)


def kernel(loc_data, conf_data, prior_data):
    conf_t = (conf_data.reshape(B, P, NCLS)
              .transpose(0, 2, 1)[:, 1:, :]
              .reshape(NPROB, P))
    loc4 = loc_data.reshape(B * P, 4)
    dx = loc4[:, 0]
    dy = loc4[:, 1]
    dw = loc4[:, 2]
    dh = loc4[:, 3]
    px = prior_data[:, 0]
    py = prior_data[:, 1]
    pw = prior_data[:, 2]
    ph = prior_data[:, 3]
    out = _detect(conf_t, dx, dy, dw, dh, px, py, pw, ph)
    return out.reshape(B, NCLS, TOP_K, 5)


# speedup vs baseline: 8.6928x; 1.0746x over previous
import struct

import jax
import jax.numpy as jnp
from jax import lax
from jax.experimental import pallas as pl
from jax.experimental.pallas import tpu as pltpu
from jax.experimental.pallas import tpu_sc as plsc

L = 16
B = 8
P = 20000
NCLS = 21
NPROB = B * (NCLS - 1)
PV = P // L
CAP = 512
NU = CAP // L
CH = 2000
NCHUNK = P // CH
CV = CH // L
NMS_TOP_K = 400
TOP_K = 200
CONF_THRESH = 0.01
NMS_THRESH = 0.45
V0 = 0.1
V1 = 0.2


def _f32_bits(x):
    return struct.unpack("<i", struct.pack("<f", x))[0]


_B001 = _f32_bits(CONF_THRESH)
_B97 = _f32_bits(0.97)
_B98 = _f32_bits(0.98)
_B99 = _f32_bits(0.99)
_BINF = 0x7F800000

_LAYERS = [(k, j)
           for k in (2, 4, 8, 16, 32)
           for j in (k // 2, k // 4, k // 8, k // 16, k // 32) if j >= 1]


def _iota():
    return lax.iota(jnp.int32, L)


def _full_i(x):
    return jnp.full((L,), x, jnp.int32)


def _lane0(v):
    return lax.squeeze(lax.slice(v, (0,), (1,)), dimensions=(0,))


def _popcnt(m):
    return _lane0(plsc.all_reduce_population_count(m))


def _detect_body(conf_hbm, dx_hbm, dy_hbm, dw_hbm, dh_hbm,
                 px_hbm, py_hbm, pw_hbm, ph_hbm, out_hbm,
                 scores, x1f, y1f, x2f, y2f,
                 ldx, ldy, ldw, ldh, lpx, lpy, lpw, lph,
                 cand_s, cand_i, skey, srt_s, srt_i,
                 x1c, y1c, x2c, y2c, ac, supp, outbuf):
    info = plsc.get_sparse_core_info()
    nc = info.num_cores
    wid = lax.axis_index("s") * nc + lax.axis_index("c")
    nper = NPROB // (nc * info.num_subcores)
    b = (wid * nper) // (NCLS - 1)
    iota = _iota()

    def count_gt(bits):
        thr = lax.bitcast_convert_type(_full_i(bits), jnp.float32)

        def cbody(k, acc):
            a0, a1 = acc
            a0 = a0 + plsc.all_reduce_population_count(
                scores[pl.ds(2 * k * L, L)] > thr)
            a1 = a1 + plsc.all_reduce_population_count(
                scores[pl.ds((2 * k + 1) * L, L)] > thr)
            return a0, a1

        z = jnp.zeros((L,), jnp.int32)
        a0, a1 = lax.fori_loop(0, PV // 2, cbody, (z, z))
        return _lane0(a0) + _lane0(a1)

    def zero_outbuf():
        def zb(k, _):
            outbuf[pl.ds(k * L, L)] = jnp.zeros((L,), jnp.float32)
            return 0
        lax.fori_loop(0, outbuf.shape[0] // L, zb, 0)

    @pl.when(wid < B)
    def _():
        zero_outbuf()
        pltpu.sync_copy(outbuf.at[pl.ds(0, TOP_K * 5)],
                        out_hbm.at[pl.ds(wid * NCLS * TOP_K * 5, TOP_K * 5)])

    def chunk(ci, _):
        off = b * P + ci * CH
        poff = ci * CH
        pltpu.sync_copy(dx_hbm.at[pl.ds(off, CH)], ldx)
        pltpu.sync_copy(dy_hbm.at[pl.ds(off, CH)], ldy)
        pltpu.sync_copy(dw_hbm.at[pl.ds(off, CH)], ldw)
        pltpu.sync_copy(dh_hbm.at[pl.ds(off, CH)], ldh)
        pltpu.sync_copy(px_hbm.at[pl.ds(poff, CH)], lpx)
        pltpu.sync_copy(py_hbm.at[pl.ds(poff, CH)], lpy)
        pltpu.sync_copy(pw_hbm.at[pl.ds(poff, CH)], lpw)
        pltpu.sync_copy(ph_hbm.at[pl.ds(poff, CH)], lph)

        def dec(k, _):
            sl = pl.ds(k * L, L)
            dx = ldx[sl]
            dy = ldy[sl]
            dw = ldw[sl]
            dh = ldh[sl]
            px = lpx[sl]
            py = lpy[sl]
            pw = lpw[sl]
            ph = lph[sl]
            cx = px + dx * jnp.float32(V0) * pw
            cy = py + dy * jnp.float32(V0) * ph
            w = pw * jnp.exp(dw * jnp.float32(V1))
            h = ph * jnp.exp(dh * jnp.float32(V1))
            x1 = cx - w / 2.0
            y1 = cy - h / 2.0
            osl = pl.ds(poff + k * L, L)
            x1f[osl] = x1
            y1f[osl] = y1
            x2f[osl] = x1 + w
            y2f[osl] = y1 + h
            return 0

        lax.fori_loop(0, CV, dec, 0)
        return 0

    lax.fori_loop(0, NCHUNK, chunk, 0)

    def problem(q, _):
        prob = wid * nper + q
        c = prob % (NCLS - 1)

        pltpu.sync_copy(conf_hbm.at[prob], scores)

        t97 = lax.bitcast_convert_type(_full_i(_B97), jnp.float32)
        t98 = lax.bitcast_convert_type(_full_i(_B98), jnp.float32)

        def p1(k, acc):
            a0, a1, a2, a3 = acc
            s0 = scores[pl.ds(2 * k * L, L)]
            s1 = scores[pl.ds((2 * k + 1) * L, L)]
            a0 = a0 + plsc.all_reduce_population_count(s0 > t97)
            a1 = a1 + plsc.all_reduce_population_count(s0 > t98)
            a2 = a2 + plsc.all_reduce_population_count(s1 > t97)
            a3 = a3 + plsc.all_reduce_population_count(s1 > t98)
            return a0, a1, a2, a3

        z = jnp.zeros((L,), jnp.int32)
        a0, a1, a2, a3 = lax.fori_loop(0, PV // 2, p1, (z, z, z, z))
        c97 = _lane0(a0) + _lane0(a2)
        c98 = _lane0(a1) + _lane0(a3)

        K = jnp.int32(NMS_TOP_K)
        c001 = lax.cond(c97 < K, lambda: count_gt(jnp.int32(_B001)),
                        lambda: jnp.int32(0x7FFFFFFF))
        lo_b = jnp.where(c98 >= K, jnp.int32(_B98),
                         jnp.where(c97 >= K, jnp.int32(_B97),
                                   jnp.int32(_B001)))
        cnt_lo = jnp.where(c98 >= K, c98, jnp.where(c97 >= K, c97, c001))
        hi_b = jnp.where(c98 >= K, jnp.int32(_B99),
                         jnp.where(c97 >= K, jnp.int32(_B98),
                                   jnp.int32(_B97)))

        def s_cond(st):
            lo, hi, cl = st
            return (cl > jnp.int32(CAP)) & (hi - lo > 1)

        def s_body(st):
            lo, hi, cl = st
            mid = lo + (hi - lo) // 2
            cm = count_gt(mid)
            lo2 = jnp.where(cm >= K, mid, lo)
            cl2 = jnp.where(cm >= K, cm, cl)
            hi2 = jnp.where(cm >= K, hi, mid)
            return lo2, hi2, cl2

        lo_b, hi_b, cnt_lo = lax.while_loop(s_cond, s_body,
                                            (lo_b, hi_b, cnt_lo))
        thr = lax.bitcast_convert_type(_full_i(lo_b), jnp.float32)

        def coll1(k, st):
            cnt, mnb, mxb = st
            s = scores[pl.ds(k * L, L)]
            m = s > thr

            @pl.when(cnt < jnp.int32(CAP))
            def _():
                plsc.store_compressed(cand_s.at[pl.ds(cnt, L)], s, mask=m)
                plsc.store_compressed(cand_i.at[pl.ds(cnt, L)],
                                      iota + k * L, mask=m)

            sb = lax.bitcast_convert_type(s, jnp.int32)
            mnb = jnp.minimum(mnb, jnp.where(m, sb, jnp.int32(0x7F7FFFFF)))
            mxb = jnp.maximum(mxb, jnp.where(m, sb, jnp.int32(0)))
            cnt = cnt + _popcnt(m)
            return cnt, mnb, mxb

        def coll(k, st):
            return coll1(2 * k + 1, coll1(2 * k, st))

        cnt, mnb_v, mxb_v = lax.fori_loop(
            0, PV // 2, coll,
            (jnp.int32(0), _full_i(0x7F7FFFFF), _full_i(0)))
        M = jnp.minimum(cnt, jnp.int32(CAP))
        mn_s, _d0 = plsc.sort_key_val(mnb_v, mnb_v)
        mx_s, _d1 = plsc.sort_key_val(mxb_v, mxb_v, descending=True)
        mnb = _lane0(mn_s)
        mxb = _lane0(mx_s)
        rng = mxb - mnb

        def sh_cond(sh):
            return lax.shift_right_arithmetic(rng, sh) >= jnp.int32(1 << 22)

        shift = lax.while_loop(sh_cond, lambda sh: sh + 1, jnp.int32(0))
        shv = _full_i(0) + shift

        def mkkey(k, _):
            base = k * L
            ids = iota + base
            valid = ids < M
            sb = lax.bitcast_convert_type(cand_s[pl.ds(base, L)], jnp.int32)
            d = lax.shift_right_arithmetic(sb - mnb, shv)
            key = jnp.bitwise_or(lax.shift_left(d, _full_i(9)),
                                 jnp.int32(CAP - 1) - ids)
            key = jnp.where(valid, key + 1, 0)
            sk, _sv = plsc.sort_key_val(key, key, descending=True)
            skey[pl.ds(base, L)] = sk
            return 0

        lax.fori_loop(0, NU, mkkey, 0)

        for (kk, jj) in _LAYERS:
            def net(t, _, kk=kk, jj=jj):
                sh = jj.bit_length() - 1
                i = ((t >> sh) << (sh + 1)) | (t & (jj - 1))
                l = i + jj
                a = skey[pl.ds(i * L, L)]
                bb = skey[pl.ds(l * L, L)]
                rb = lax.rev(bb, (0,))
                hi = jnp.maximum(a, rb)
                lo = jnp.minimum(a, rb)
                hi_s, _h = plsc.sort_key_val(hi, hi, descending=True)
                lo_s, _l = plsc.sort_key_val(lo, lo, descending=True)
                maxfirst = jnp.bitwise_and(i, jnp.int32(kk)) == 0
                skey[pl.ds(i * L, L)] = jnp.where(maxfirst, hi_s, lo_s)
                skey[pl.ds(l * L, L)] = jnp.where(maxfirst, lo_s, hi_s)
                return 0

            lax.fori_loop(0, NU // 2, net, 0)

        def unp(k, _):
            base = k * L
            sk = skey[pl.ds(base, L)]
            ordv = jnp.where(sk > 0,
                             jnp.int32(CAP - 1) -
                             jnp.bitwise_and(sk - 1, jnp.int32(CAP - 1)),
                             0)
            srt_s[pl.ds(base, L)] = plsc.load_gather(cand_s, [ordv])
            pi = plsc.load_gather(cand_i, [ordv])
            pi = jnp.minimum(jnp.maximum(pi, 0), jnp.int32(P - 1))
            srt_i[pl.ds(base, L)] = pi
            x1 = plsc.load_gather(x1f, [pi])
            y1 = plsc.load_gather(y1f, [pi])
            x2 = plsc.load_gather(x2f, [pi])
            y2 = plsc.load_gather(y2f, [pi])
            x1c[pl.ds(base, L)] = x1
            y1c[pl.ds(base, L)] = y1
            x2c[pl.ds(base, L)] = x2
            y2c[pl.ds(base, L)] = y2
            ac[pl.ds(base, L)] = (x2 - x1) * (y2 - y1)
            supp[pl.ds(base, L)] = jnp.zeros((L,), jnp.float32)
            return 0

        lax.fori_loop(0, NU, unp, 0)

        zero_outbuf()
        Mc = jnp.minimum(M, jnp.int32(NMS_TOP_K))

        def splat(ref, i):
            return plsc.load_gather(ref, [_full_i(0) + i])

        def compact(i, mc):
            blk0 = (i + L) // L
            dst0 = blk0 * L
            nb_old = (mc + (L - 1)) // L

            def cblk(kb2, dst):
                sl = pl.ds(kb2 * L, L)
                jids = iota + kb2 * L
                alive = (supp[sl] == 0.0) & (jids < mc)
                sv = srt_s[sl]
                x1v = x1c[sl]
                y1v = y1c[sl]
                x2v = x2c[sl]
                y2v = y2c[sl]
                av = ac[sl]
                plsc.store_compressed(srt_s.at[pl.ds(dst, L)], sv, mask=alive)
                plsc.store_compressed(x1c.at[pl.ds(dst, L)], x1v, mask=alive)
                plsc.store_compressed(y1c.at[pl.ds(dst, L)], y1v, mask=alive)
                plsc.store_compressed(x2c.at[pl.ds(dst, L)], x2v, mask=alive)
                plsc.store_compressed(y2c.at[pl.ds(dst, L)], y2v, mask=alive)
                plsc.store_compressed(ac.at[pl.ds(dst, L)], av, mask=alive)
                return dst + _popcnt(alive)

            dst = lax.fori_loop(blk0, nb_old, cblk, jnp.int32(dst0))

            def zblk(kb2, _):
                supp[pl.ds(kb2 * L, L)] = jnp.zeros((L,), jnp.float32)
                return 0

            lax.fori_loop(blk0, (dst + (L - 1)) // L, zblk, 0)
            return dst

        def n_cond(st):
            i, nk, mc, nextc = st
            return (i < mc) & (nk < jnp.int32(TOP_K))

        def n_body(st):
            i, nk, mc, nextc = st

            def do_c(opn):
                i, mc = opn
                return compact(i, mc), i + jnp.int32(64)

            def no_c(opn):
                i, mc = opn
                return mc, nextc

            mc, nextc = lax.cond((i >= nextc) & (mc - i > jnp.int32(48)),
                                 do_c, no_c, (i, mc))
            sup_i = _lane0(splat(supp, i))

            def keep(opn):
                i, nk, mc = opn
                NB = (mc + (L - 1)) // L
                sv = splat(srt_s, i)
                x1i = splat(x1c, i)
                y1i = splat(y1c, i)
                x2i = splat(x2c, i)
                y2i = splat(y2c, i)
                ai = splat(ac, i)
                oidx = nk * 5 + iota
                val = sv
                val = jnp.where(iota == 1, x1i, val)
                val = jnp.where(iota == 2, y1i, val)
                val = jnp.where(iota == 3, x2i, val)
                val = jnp.where(iota == 4, y2i, val)
                plsc.store_scatter(outbuf, [oidx], val, mask=iota < 5)

                iv = _full_i(0) + i
                kb = i // L

                def sup_blk(k, _):
                    base = k * L
                    jids = iota + base
                    bx1 = x1c[pl.ds(base, L)]
                    by1 = y1c[pl.ds(base, L)]
                    bx2 = x2c[pl.ds(base, L)]
                    by2 = y2c[pl.ds(base, L)]
                    ba = ac[pl.ds(base, L)]
                    ix1 = jnp.maximum(x1i, bx1)
                    iy1 = jnp.maximum(y1i, by1)
                    ix2 = jnp.minimum(x2i, bx2)
                    iy2 = jnp.minimum(y2i, by2)
                    inter = (jnp.maximum(ix2 - ix1, 0.0) *
                             jnp.maximum(iy2 - iy1, 0.0))
                    union = ai + ba - inter
                    iou = inter / jnp.maximum(union, jnp.float32(1e-12))
                    sup = (iou > jnp.float32(NMS_THRESH)) & (jids > iv)
                    sp = supp[pl.ds(base, L)]
                    supp[pl.ds(base, L)] = jnp.where(sup, 1.0, sp)
                    return 0

                lax.fori_loop(kb, NB, sup_blk, 0)
                return i + 1, nk + 1

            def skip(opn):
                i, nk, mc = opn
                return i + 1, nk

            i2, nk2 = lax.cond(sup_i == 0.0, keep, skip, (i, nk, mc))
            return i2, nk2, mc, nextc

        lax.while_loop(n_cond, n_body,
                       (jnp.int32(0), jnp.int32(0), Mc, jnp.int32(64)))

        off = (b * NCLS + c + 1) * (TOP_K * 5)
        pltpu.sync_copy(outbuf.at[pl.ds(0, TOP_K * 5)],
                        out_hbm.at[pl.ds(off, TOP_K * 5)])
        return 0

    lax.fori_loop(0, nper, problem, 0)


@jax.jit
def _detect(conf_t, dx, dy, dw, dh, px, py, pw, ph):
    mesh = plsc.VectorSubcoreMesh(core_axis_name="c", subcore_axis_name="s")
    f = pl.kernel(
        _detect_body,
        out_type=jax.ShapeDtypeStruct((B * NCLS * TOP_K * 5,), jnp.float32),
        mesh=mesh,
        compiler_params=pltpu.CompilerParams(needs_layout_passes=False,
                                             use_tc_tiling_on_sc=False),
        scratch_types=[
            pltpu.VMEM((P,), jnp.float32),
            pltpu.VMEM((P,), jnp.float32),
            pltpu.VMEM((P,), jnp.float32),
            pltpu.VMEM((P,), jnp.float32),
            pltpu.VMEM((P,), jnp.float32),
            pltpu.VMEM((CH,), jnp.float32),
            pltpu.VMEM((CH,), jnp.float32),
            pltpu.VMEM((CH,), jnp.float32),
            pltpu.VMEM((CH,), jnp.float32),
            pltpu.VMEM((CH,), jnp.float32),
            pltpu.VMEM((CH,), jnp.float32),
            pltpu.VMEM((CH,), jnp.float32),
            pltpu.VMEM((CH,), jnp.float32),
            pltpu.VMEM((CAP + L,), jnp.float32),
            pltpu.VMEM((CAP + L,), jnp.int32),
            pltpu.VMEM((CAP,), jnp.int32),
            pltpu.VMEM((CAP,), jnp.float32),
            pltpu.VMEM((CAP,), jnp.int32),
            pltpu.VMEM((CAP,), jnp.float32),
            pltpu.VMEM((CAP,), jnp.float32),
            pltpu.VMEM((CAP,), jnp.float32),
            pltpu.VMEM((CAP,), jnp.float32),
            pltpu.VMEM((CAP,), jnp.float32),
            pltpu.VMEM((CAP,), jnp.float32),
            pltpu.VMEM((TOP_K * 5 + L - (TOP_K * 5) % L,), jnp.float32),
        ],
    )
    return f(conf_t, dx, dy, dw, dh, px, py, pw, ph)


def kernel(loc_data, conf_data, prior_data):
    conf_t = (conf_data.reshape(B, P, NCLS)
              .transpose(0, 2, 1)[:, 1:, :]
              .reshape(NPROB, P))
    loc4 = loc_data.reshape(B * P, 4)
    dx = loc4[:, 0]
    dy = loc4[:, 1]
    dw = loc4[:, 2]
    dh = loc4[:, 3]
    px = prior_data[:, 0]
    py = prior_data[:, 1]
    pw = prior_data[:, 2]
    ph = prior_data[:, 3]
    out = _detect(conf_t, dx, dy, dw, dh, px, py, pw, ph)
    return out.reshape(B, NCLS, TOP_K, 5)


# speedup vs baseline: 8.8278x; 1.0155x over previous
import struct

import jax
import jax.numpy as jnp
from jax import lax
from jax.experimental import pallas as pl
from jax.experimental.pallas import tpu as pltpu
from jax.experimental.pallas import tpu_sc as plsc

L = 16
B = 8
P = 20000
NCLS = 21
NPROB = B * (NCLS - 1)
PV = P // L
CAP = 512
CAPBIG = 1024
NU = CAP // L
CH = 2000
NCHUNK = P // CH
CV = CH // L
NMS_TOP_K = 400
TOP_K = 200
CONF_THRESH = 0.01
NMS_THRESH = 0.45
V0 = 0.1
V1 = 0.2


def _f32_bits(x):
    return struct.unpack("<i", struct.pack("<f", x))[0]


_B001 = _f32_bits(CONF_THRESH)
_B97 = _f32_bits(0.97)
_B98 = _f32_bits(0.98)
_B99 = _f32_bits(0.99)
_BINF = 0x7F800000

_LAYERS = [(k, j)
           for k in (2, 4, 8, 16, 32)
           for j in (k // 2, k // 4, k // 8, k // 16, k // 32) if j >= 1]


def _iota():
    return lax.iota(jnp.int32, L)


def _full_i(x):
    return jnp.full((L,), x, jnp.int32)


def _lane0(v):
    return lax.squeeze(lax.slice(v, (0,), (1,)), dimensions=(0,))


def _popcnt(m):
    return _lane0(plsc.all_reduce_population_count(m))


def _detect_body(conf_hbm, dx_hbm, dy_hbm, dw_hbm, dh_hbm,
                 px_hbm, py_hbm, pw_hbm, ph_hbm, out_hbm,
                 scores, x1f, y1f, x2f, y2f,
                 ldx, ldy, ldw, ldh, lpx, lpy, lpw, lph,
                 cand_s, cand_i, skey, srt_s, srt_i,
                 x1c, y1c, x2c, y2c, ac, supp, outbuf):
    info = plsc.get_sparse_core_info()
    nc = info.num_cores
    wid = lax.axis_index("s") * nc + lax.axis_index("c")
    nper = NPROB // (nc * info.num_subcores)
    b = (wid * nper) // (NCLS - 1)
    iota = _iota()

    def count_gt(bits):
        thr = lax.bitcast_convert_type(_full_i(bits), jnp.float32)

        def cbody(k, acc):
            a0, a1 = acc
            a0 = a0 + plsc.all_reduce_population_count(
                scores[pl.ds(2 * k * L, L)] > thr)
            a1 = a1 + plsc.all_reduce_population_count(
                scores[pl.ds((2 * k + 1) * L, L)] > thr)
            return a0, a1

        z = jnp.zeros((L,), jnp.int32)
        a0, a1 = lax.fori_loop(0, PV // 2, cbody, (z, z))
        return _lane0(a0) + _lane0(a1)

    def zero_outbuf():
        def zb(k, _):
            outbuf[pl.ds(k * L, L)] = jnp.zeros((L,), jnp.float32)
            return 0
        lax.fori_loop(0, outbuf.shape[0] // L, zb, 0)

    @pl.when(wid < B)
    def _():
        zero_outbuf()
        pltpu.sync_copy(outbuf.at[pl.ds(0, TOP_K * 5)],
                        out_hbm.at[pl.ds(wid * NCLS * TOP_K * 5, TOP_K * 5)])

    def chunk(ci, _):
        off = b * P + ci * CH
        poff = ci * CH
        pltpu.sync_copy(dx_hbm.at[pl.ds(off, CH)], ldx)
        pltpu.sync_copy(dy_hbm.at[pl.ds(off, CH)], ldy)
        pltpu.sync_copy(dw_hbm.at[pl.ds(off, CH)], ldw)
        pltpu.sync_copy(dh_hbm.at[pl.ds(off, CH)], ldh)
        pltpu.sync_copy(px_hbm.at[pl.ds(poff, CH)], lpx)
        pltpu.sync_copy(py_hbm.at[pl.ds(poff, CH)], lpy)
        pltpu.sync_copy(pw_hbm.at[pl.ds(poff, CH)], lpw)
        pltpu.sync_copy(ph_hbm.at[pl.ds(poff, CH)], lph)

        def dec(k, _):
            sl = pl.ds(k * L, L)
            dx = ldx[sl]
            dy = ldy[sl]
            dw = ldw[sl]
            dh = ldh[sl]
            px = lpx[sl]
            py = lpy[sl]
            pw = lpw[sl]
            ph = lph[sl]
            cx = px + dx * jnp.float32(V0) * pw
            cy = py + dy * jnp.float32(V0) * ph
            w = pw * jnp.exp(dw * jnp.float32(V1))
            h = ph * jnp.exp(dh * jnp.float32(V1))
            x1 = cx - w / 2.0
            y1 = cy - h / 2.0
            osl = pl.ds(poff + k * L, L)
            x1f[osl] = x1
            y1f[osl] = y1
            x2f[osl] = x1 + w
            y2f[osl] = y1 + h
            return 0

        lax.fori_loop(0, CV, dec, 0)
        return 0

    lax.fori_loop(0, NCHUNK, chunk, 0)

    def problem(q, _):
        prob = wid * nper + q
        c = prob % (NCLS - 1)

        pltpu.sync_copy(conf_hbm.at[prob], scores)

        K = jnp.int32(NMS_TOP_K)
        t97 = lax.bitcast_convert_type(_full_i(_B97), jnp.float32)

        def collA1(k, st):
            cnt, mnb, mxb = st
            s = scores[pl.ds(k * L, L)]
            m = s > t97

            @pl.when(cnt < jnp.int32(CAPBIG))
            def _():
                plsc.store_compressed(cand_s.at[pl.ds(cnt, L)], s, mask=m)
                plsc.store_compressed(cand_i.at[pl.ds(cnt, L)],
                                      iota + k * L, mask=m)

            sb = lax.bitcast_convert_type(s, jnp.int32)
            mnb = jnp.minimum(mnb, jnp.where(m, sb, jnp.int32(0x7F7FFFFF)))
            mxb = jnp.maximum(mxb, jnp.where(m, sb, jnp.int32(0)))
            cnt = cnt + _popcnt(m)
            return cnt, mnb, mxb

        def collA(k, st):
            return collA1(2 * k + 1, collA1(2 * k, st))

        cntA, mnb_vA, mxb_vA = lax.fori_loop(
            0, PV // 2, collA,
            (jnp.int32(0), _full_i(0x7F7FFFFF), _full_i(0)))
        mnA_s, _e0 = plsc.sort_key_val(mnb_vA, mnb_vA)
        mxA_s, _e1 = plsc.sort_key_val(mxb_vA, mxb_vA, descending=True)
        mnbA = _lane0(mnA_s)
        mxbA = _lane0(mxA_s)

        def cand_count_gt(bits, nblk):
            thr2 = lax.bitcast_convert_type(_full_i(bits), jnp.float32)

            def cb(k, acc):
                m = (cand_s[pl.ds(k * L, L)] > thr2) & (iota + k * L < cntA)
                return acc + plsc.all_reduce_population_count(m)

            return _lane0(lax.fori_loop(0, nblk, cb, jnp.zeros((L,),
                                                               jnp.int32)))

        def fast_path(_):
            nblk = (cntA + (L - 1)) // L

            def s_cond(st):
                lo, hi, cl = st
                return (cl > jnp.int32(CAP)) & (hi - lo > 1)

            def s_body(st):
                lo, hi, cl = st
                mid = lo + (hi - lo) // 2
                cm = cand_count_gt(mid, nblk)
                lo2 = jnp.where(cm >= K, mid, lo)
                cl2 = jnp.where(cm >= K, cm, cl)
                hi2 = jnp.where(cm >= K, hi, mid)
                return lo2, hi2, cl2

            lo_b, _hb, cnt2 = lax.while_loop(
                s_cond, s_body, (jnp.int32(_B97), mxbA, cntA))

            def do_compact(opn):
                lo_b, = opn
                thr2 = lax.bitcast_convert_type(_full_i(lo_b), jnp.float32)

                def fc(k, dst):
                    sl = pl.ds(k * L, L)
                    sv = cand_s[sl]
                    iv = cand_i[sl]
                    m = (sv > thr2) & (iota + k * L < cntA)

                    @pl.when(dst < jnp.int32(CAP))
                    def _():
                        plsc.store_compressed(cand_s.at[pl.ds(dst, L)],
                                              sv, mask=m)
                        plsc.store_compressed(cand_i.at[pl.ds(dst, L)],
                                              iv, mask=m)

                    return dst + _popcnt(m)

                return lax.fori_loop(0, nblk, fc, jnp.int32(0))

            cnt = lax.cond(cntA > jnp.int32(CAP), do_compact,
                           lambda opn: cntA, (lo_b,))
            return cnt

        def slow_path(_):
            c001 = count_gt(jnp.int32(_B001))
            lo_b = jnp.where(cntA >= K, jnp.int32(_B97), jnp.int32(_B001))
            cnt_lo = jnp.where(cntA >= K, cntA, c001)
            hi_b = jnp.where(cntA >= K, jnp.int32(_BINF), jnp.int32(_B97))

            def s_cond(st):
                lo, hi, cl = st
                return (cl > jnp.int32(CAP)) & (hi - lo > 1)

            def s_body(st):
                lo, hi, cl = st
                mid = lo + (hi - lo) // 2
                cm = count_gt(mid)
                lo2 = jnp.where(cm >= K, mid, lo)
                cl2 = jnp.where(cm >= K, cm, cl)
                hi2 = jnp.where(cm >= K, hi, mid)
                return lo2, hi2, cl2

            lo_b, hi_b, cnt_lo = lax.while_loop(s_cond, s_body,
                                                (lo_b, hi_b, cnt_lo))
            thr = lax.bitcast_convert_type(_full_i(lo_b), jnp.float32)

            def coll1(k, cnt):
                s = scores[pl.ds(k * L, L)]
                m = s > thr

                @pl.when(cnt < jnp.int32(CAP))
                def _():
                    plsc.store_compressed(cand_s.at[pl.ds(cnt, L)],
                                          s, mask=m)
                    plsc.store_compressed(cand_i.at[pl.ds(cnt, L)],
                                          iota + k * L, mask=m)

                return cnt + _popcnt(m)

            def coll(k, cnt):
                return coll1(2 * k + 1, coll1(2 * k, cnt))

            cnt = lax.fori_loop(0, PV // 2, coll, jnp.int32(0))

            def mm(k, st):
                mnb, mxb = st
                s = cand_s[pl.ds(k * L, L)]
                ids = iota + k * L
                m = ids < jnp.minimum(cnt, jnp.int32(CAP))
                sb = lax.bitcast_convert_type(s, jnp.int32)
                mnb = jnp.minimum(mnb, jnp.where(m, sb,
                                                 jnp.int32(0x7F7FFFFF)))
                mxb = jnp.maximum(mxb, jnp.where(m, sb, jnp.int32(0)))
                return mnb, mxb

            mn_v, mx_v = lax.fori_loop(0, NU, mm,
                                       (_full_i(0x7F7FFFFF), _full_i(0)))
            mn_s, _d0 = plsc.sort_key_val(mn_v, mn_v)
            mx_s, _d1 = plsc.sort_key_val(mx_v, mx_v, descending=True)
            return cnt, _lane0(mn_s), _lane0(mx_s)

        def fast_wrap(opn):
            return fast_path(opn), mnbA, mxbA

        use_fast = (cntA >= K) & (cntA <= jnp.int32(CAPBIG))
        cnt, mnb, mxb = lax.cond(use_fast, fast_wrap, slow_path, ())
        M = jnp.minimum(cnt, jnp.int32(CAP))
        rng = mxb - mnb

        def sh_cond(sh):
            return lax.shift_right_arithmetic(rng, sh) >= jnp.int32(1 << 22)

        shift = lax.while_loop(sh_cond, lambda sh: sh + 1, jnp.int32(0))
        shv = _full_i(0) + shift

        def mkkey(k, _):
            base = k * L
            ids = iota + base
            valid = ids < M
            sb = lax.bitcast_convert_type(cand_s[pl.ds(base, L)], jnp.int32)
            d = lax.shift_right_arithmetic(sb - mnb, shv)
            key = jnp.bitwise_or(lax.shift_left(d, _full_i(9)),
                                 jnp.int32(CAP - 1) - ids)
            key = jnp.where(valid, key + 1, 0)
            sk, _sv = plsc.sort_key_val(key, key, descending=True)
            skey[pl.ds(base, L)] = sk
            return 0

        lax.fori_loop(0, NU, mkkey, 0)

        for (kk, jj) in _LAYERS:
            def net(t, _, kk=kk, jj=jj):
                sh = jj.bit_length() - 1
                i = ((t >> sh) << (sh + 1)) | (t & (jj - 1))
                l = i + jj
                a = skey[pl.ds(i * L, L)]
                bb = skey[pl.ds(l * L, L)]
                rb = lax.rev(bb, (0,))
                hi = jnp.maximum(a, rb)
                lo = jnp.minimum(a, rb)
                hi_s, _h = plsc.sort_key_val(hi, hi, descending=True)
                lo_s, _l = plsc.sort_key_val(lo, lo, descending=True)
                maxfirst = jnp.bitwise_and(i, jnp.int32(kk)) == 0
                skey[pl.ds(i * L, L)] = jnp.where(maxfirst, hi_s, lo_s)
                skey[pl.ds(l * L, L)] = jnp.where(maxfirst, lo_s, hi_s)
                return 0

            lax.fori_loop(0, NU // 2, net, 0)

        def unp(k, _):
            base = k * L
            sk = skey[pl.ds(base, L)]
            ordv = jnp.where(sk > 0,
                             jnp.int32(CAP - 1) -
                             jnp.bitwise_and(sk - 1, jnp.int32(CAP - 1)),
                             0)
            srt_s[pl.ds(base, L)] = plsc.load_gather(cand_s, [ordv])
            pi = plsc.load_gather(cand_i, [ordv])
            pi = jnp.minimum(jnp.maximum(pi, 0), jnp.int32(P - 1))
            srt_i[pl.ds(base, L)] = pi
            x1 = plsc.load_gather(x1f, [pi])
            y1 = plsc.load_gather(y1f, [pi])
            x2 = plsc.load_gather(x2f, [pi])
            y2 = plsc.load_gather(y2f, [pi])
            x1c[pl.ds(base, L)] = x1
            y1c[pl.ds(base, L)] = y1
            x2c[pl.ds(base, L)] = x2
            y2c[pl.ds(base, L)] = y2
            ac[pl.ds(base, L)] = (x2 - x1) * (y2 - y1)
            supp[pl.ds(base, L)] = jnp.zeros((L,), jnp.float32)
            return 0

        lax.fori_loop(0, NU, unp, 0)

        zero_outbuf()
        Mc = jnp.minimum(M, jnp.int32(NMS_TOP_K))

        def splat(ref, i):
            return plsc.load_gather(ref, [_full_i(0) + i])

        def compact(i, mc):
            blk0 = (i + L) // L
            dst0 = blk0 * L
            nb_old = (mc + (L - 1)) // L

            def cblk(kb2, dst):
                sl = pl.ds(kb2 * L, L)
                jids = iota + kb2 * L
                alive = (supp[sl] == 0.0) & (jids < mc)
                sv = srt_s[sl]
                x1v = x1c[sl]
                y1v = y1c[sl]
                x2v = x2c[sl]
                y2v = y2c[sl]
                av = ac[sl]
                plsc.store_compressed(srt_s.at[pl.ds(dst, L)], sv, mask=alive)
                plsc.store_compressed(x1c.at[pl.ds(dst, L)], x1v, mask=alive)
                plsc.store_compressed(y1c.at[pl.ds(dst, L)], y1v, mask=alive)
                plsc.store_compressed(x2c.at[pl.ds(dst, L)], x2v, mask=alive)
                plsc.store_compressed(y2c.at[pl.ds(dst, L)], y2v, mask=alive)
                plsc.store_compressed(ac.at[pl.ds(dst, L)], av, mask=alive)
                return dst + _popcnt(alive)

            dst = lax.fori_loop(blk0, nb_old, cblk, jnp.int32(dst0))

            def zblk(kb2, _):
                supp[pl.ds(kb2 * L, L)] = jnp.zeros((L,), jnp.float32)
                return 0

            lax.fori_loop(blk0, (dst + (L - 1)) // L, zblk, 0)
            return dst

        def n_cond(st):
            i, nk, mc, nextc = st
            return (i < mc) & (nk < jnp.int32(TOP_K))

        def n_body(st):
            i, nk, mc, nextc = st

            def do_c(opn):
                i, mc = opn
                return compact(i, mc), i + jnp.int32(64)

            def no_c(opn):
                i, mc = opn
                return mc, nextc

            mc, nextc = lax.cond((i >= nextc) & (mc - i > jnp.int32(48)),
                                 do_c, no_c, (i, mc))
            sup_i = _lane0(splat(supp, i))

            def keep(opn):
                i, nk, mc = opn
                NB = (mc + (L - 1)) // L
                sv = splat(srt_s, i)
                x1i = splat(x1c, i)
                y1i = splat(y1c, i)
                x2i = splat(x2c, i)
                y2i = splat(y2c, i)
                ai = splat(ac, i)
                oidx = nk * 5 + iota
                val = sv
                val = jnp.where(iota == 1, x1i, val)
                val = jnp.where(iota == 2, y1i, val)
                val = jnp.where(iota == 3, x2i, val)
                val = jnp.where(iota == 4, y2i, val)
                plsc.store_scatter(outbuf, [oidx], val, mask=iota < 5)

                iv = _full_i(0) + i
                kb = i // L

                def sup_blk(k, _):
                    base = k * L
                    jids = iota + base
                    bx1 = x1c[pl.ds(base, L)]
                    by1 = y1c[pl.ds(base, L)]
                    bx2 = x2c[pl.ds(base, L)]
                    by2 = y2c[pl.ds(base, L)]
                    ba = ac[pl.ds(base, L)]
                    ix1 = jnp.maximum(x1i, bx1)
                    iy1 = jnp.maximum(y1i, by1)
                    ix2 = jnp.minimum(x2i, bx2)
                    iy2 = jnp.minimum(y2i, by2)
                    inter = (jnp.maximum(ix2 - ix1, 0.0) *
                             jnp.maximum(iy2 - iy1, 0.0))
                    union = ai + ba - inter
                    iou = inter / jnp.maximum(union, jnp.float32(1e-12))
                    sup = (iou > jnp.float32(NMS_THRESH)) & (jids > iv)
                    sp = supp[pl.ds(base, L)]
                    supp[pl.ds(base, L)] = jnp.where(sup, 1.0, sp)
                    return 0

                lax.fori_loop(kb, NB, sup_blk, 0)
                return i + 1, nk + 1

            def skip(opn):
                i, nk, mc = opn
                return i + 1, nk

            i2, nk2 = lax.cond(sup_i == 0.0, keep, skip, (i, nk, mc))
            return i2, nk2, mc, nextc

        lax.while_loop(n_cond, n_body,
                       (jnp.int32(0), jnp.int32(0), Mc, jnp.int32(64)))

        off = (b * NCLS + c + 1) * (TOP_K * 5)
        pltpu.sync_copy(outbuf.at[pl.ds(0, TOP_K * 5)],
                        out_hbm.at[pl.ds(off, TOP_K * 5)])
        return 0

    lax.fori_loop(0, nper, problem, 0)


@jax.jit
def _detect(conf_t, dx, dy, dw, dh, px, py, pw, ph):
    mesh = plsc.VectorSubcoreMesh(core_axis_name="c", subcore_axis_name="s")
    f = pl.kernel(
        _detect_body,
        out_type=jax.ShapeDtypeStruct((B * NCLS * TOP_K * 5,), jnp.float32),
        mesh=mesh,
        compiler_params=pltpu.CompilerParams(needs_layout_passes=False,
                                             use_tc_tiling_on_sc=False),
        scratch_types=[
            pltpu.VMEM((P,), jnp.float32),
            pltpu.VMEM((P,), jnp.float32),
            pltpu.VMEM((P,), jnp.float32),
            pltpu.VMEM((P,), jnp.float32),
            pltpu.VMEM((P,), jnp.float32),
            pltpu.VMEM((CH,), jnp.float32),
            pltpu.VMEM((CH,), jnp.float32),
            pltpu.VMEM((CH,), jnp.float32),
            pltpu.VMEM((CH,), jnp.float32),
            pltpu.VMEM((CH,), jnp.float32),
            pltpu.VMEM((CH,), jnp.float32),
            pltpu.VMEM((CH,), jnp.float32),
            pltpu.VMEM((CH,), jnp.float32),
            pltpu.VMEM((CAPBIG + L,), jnp.float32),
            pltpu.VMEM((CAPBIG + L,), jnp.int32),
            pltpu.VMEM((CAP,), jnp.int32),
            pltpu.VMEM((CAP,), jnp.float32),
            pltpu.VMEM((CAP,), jnp.int32),
            pltpu.VMEM((CAP,), jnp.float32),
            pltpu.VMEM((CAP,), jnp.float32),
            pltpu.VMEM((CAP,), jnp.float32),
            pltpu.VMEM((CAP,), jnp.float32),
            pltpu.VMEM((CAP,), jnp.float32),
            pltpu.VMEM((CAP,), jnp.float32),
            pltpu.VMEM((TOP_K * 5 + L - (TOP_K * 5) % L,), jnp.float32),
        ],
    )
    return f(conf_t, dx, dy, dw, dh, px, py, pw, ph)


def kernel(loc_data, conf_data, prior_data):
    conf_t = (conf_data.reshape(B, P, NCLS)
              .transpose(0, 2, 1)[:, 1:, :]
              .reshape(NPROB, P))
    loc4 = loc_data.reshape(B * P, 4)
    dx = loc4[:, 0]
    dy = loc4[:, 1]
    dw = loc4[:, 2]
    dh = loc4[:, 3]
    px = prior_data[:, 0]
    py = prior_data[:, 1]
    pw = prior_data[:, 2]
    ph = prior_data[:, 3]
    out = _detect(conf_t, dx, dy, dw, dh, px, py, pw, ph)
    return out.reshape(B, NCLS, TOP_K, 5)
